# baseline (speedup 1.0000x reference)
import jax
import jax.numpy as jnp
from jax.experimental import pallas as pl

_D_MODEL = 2048
_E = 16
_BLOCK = 512


def _router_body(x_ref, w_ref, probs_ref, tw_ref, te_ref):
    logits = jnp.dot(x_ref[...], w_ref[...], preferred_element_type=jnp.float32)
    m = jnp.max(logits, axis=-1, keepdims=True)
    e = jnp.exp(logits - m)
    s = jnp.sum(e, axis=-1, keepdims=True)
    probs_ref[...] = e / s

    iota = jax.lax.broadcasted_iota(jnp.int32, e.shape, 1)
    enc = (e.view(jnp.int32) & jnp.int32(~0xF)) | (jnp.int32(15) - iota)
    encf = enc.view(jnp.float32)
    m1 = jnp.max(encf, axis=-1, keepdims=True)
    masked = jnp.where(encf == m1, jnp.float32(-1.0), encf)
    m2 = jnp.max(masked, axis=-1, keepdims=True)

    i1 = jnp.int32(15) - (m1.view(jnp.int32) & jnp.int32(0xF))
    i2 = jnp.int32(15) - (m2.view(jnp.int32) & jnp.int32(0xF))
    v1 = (m1.view(jnp.int32) & jnp.int32(~0xF)).view(jnp.float32)
    v2 = (m2.view(jnp.int32) & jnp.int32(~0xF)).view(jnp.float32)
    tw_ref[...] = jnp.concatenate([v1, v2], axis=-1) / (v1 + v2)
    te_ref[...] = jnp.concatenate([i1, i2], axis=-1)


def kernel(x, W):
    n = x.shape[0]
    grid = (n // _BLOCK,)
    probs, tw, te = pl.pallas_call(
        _router_body,
        grid=grid,
        in_specs=[
            pl.BlockSpec((_BLOCK, _D_MODEL), lambda i: (i, 0)),
            pl.BlockSpec((_D_MODEL, _E), lambda i: (0, 0)),
        ],
        out_specs=[
            pl.BlockSpec((_BLOCK, _E), lambda i: (i, 0)),
            pl.BlockSpec((_BLOCK, 2), lambda i: (i, 0)),
            pl.BlockSpec((_BLOCK, 2), lambda i: (i, 0)),
        ],
        out_shape=[
            jax.ShapeDtypeStruct((n, _E), jnp.float32),
            jax.ShapeDtypeStruct((n, 2), jnp.float32),
            jax.ShapeDtypeStruct((n, 2), jnp.int32),
        ],
    )(x, W)
    return (probs, tw, te)


# speedup vs baseline: 1.5682x; 1.5682x over previous
import jax
import jax.numpy as jnp
from jax.experimental import pallas as pl

_D_MODEL = 2048
_E = 16
_BLOCK = 1024


def _router_body(x_ref, w_ref, probs_ref, tw_ref, te_ref):
    logits = jnp.dot(x_ref[...], w_ref[...], preferred_element_type=jnp.float32)
    lt = logits.T
    m = jnp.max(lt, axis=0, keepdims=True)
    e = jnp.exp(lt - m)
    s = jnp.sum(e, axis=0, keepdims=True)
    probs_ref[...] = (e * (1.0 / s)).T

    iota = jax.lax.broadcasted_iota(jnp.int32, e.shape, 0)
    enc = (e.view(jnp.int32) & jnp.int32(~0xF)) | (jnp.int32(15) - iota)
    encf = enc.view(jnp.float32)
    m1 = jnp.max(encf, axis=0, keepdims=True)
    masked = jnp.where(encf == m1, jnp.float32(-1.0), encf)
    m2 = jnp.max(masked, axis=0, keepdims=True)

    i1 = jnp.int32(15) - (m1.view(jnp.int32) & jnp.int32(0xF))
    i2 = jnp.int32(15) - (m2.view(jnp.int32) & jnp.int32(0xF))
    v1 = (m1.view(jnp.int32) & jnp.int32(~0xF)).view(jnp.float32)
    v2 = (m2.view(jnp.int32) & jnp.int32(~0xF)).view(jnp.float32)
    r = 1.0 / (v1 + v2)
    tw_ref[...] = jnp.concatenate([v1 * r, v2 * r], axis=0)
    te_ref[...] = jnp.concatenate([i1, i2], axis=0)


def kernel(x, W):
    n = x.shape[0]
    grid = (n // _BLOCK,)
    probs, twt, tet = pl.pallas_call(
        _router_body,
        grid=grid,
        in_specs=[
            pl.BlockSpec((_BLOCK, _D_MODEL), lambda i: (i, 0)),
            pl.BlockSpec((_D_MODEL, _E), lambda i: (0, 0)),
        ],
        out_specs=[
            pl.BlockSpec((_BLOCK, _E), lambda i: (i, 0)),
            pl.BlockSpec((2, _BLOCK), lambda i: (0, i)),
            pl.BlockSpec((2, _BLOCK), lambda i: (0, i)),
        ],
        out_shape=[
            jax.ShapeDtypeStruct((n, _E), jnp.float32),
            jax.ShapeDtypeStruct((2, n), jnp.float32),
            jax.ShapeDtypeStruct((2, n), jnp.int32),
        ],
    )(x, W)
    return (probs, twt.T, tet.T)
